# baseline (device time: 18234 ns/iter reference)
import jax
import jax.numpy as jnp
from jax import lax
from jax.experimental import pallas as pl
from jax.experimental.pallas import tpu as pltpu

N_DEV = 4
N_TOK = 512
D_IN = 256
D_OUT = 512
N_EXP = 16
E_LOCAL = N_EXP // N_DEV
CHUNK = N_TOK // N_DEV
HALF = CHUNK // 2
N_PEER = N_DEV - 1


def _mod(v):
    return lax.rem(v + 2 * N_DEV, N_DEV)


def kernel(x, router_W, route_idx, expert_W):
    def body(x_ref, rw_ref, idx_ref, ew_ref, out_ref,
             xv_ref, ewf_ref, ewb_ref, w_ref, rs_send_buf, rs_recv_buf,
             in_sems, rs_send_sems, rs_recv_sems, ag_send_sems, ag_recv_sems):
        my = lax.axis_index("i")
        peers = [_mod(my + k + 1) for k in range(N_PEER)]

        x_dma = pltpu.make_async_copy(x_ref, xv_ref, in_sems.at[0])
        x_dma.start()
        ew_dma = pltpu.make_async_copy(ew_ref, ewf_ref, in_sems.at[1])
        ew_dma.start()

        barrier_sem = pltpu.get_barrier_semaphore()
        for q in peers:
            pl.semaphore_signal(
                barrier_sem, inc=1,
                device_id=(q,), device_id_type=pl.DeviceIdType.MESH,
            )

        x_dma.wait()
        xf = xv_ref[:, :]
        scores = jnp.dot(xf, rw_ref[:, :], preferred_element_type=jnp.float32)
        s_max = jnp.max(scores, axis=-1, keepdims=True)
        p = jnp.exp(scores - s_max)
        probs = p / jnp.sum(p, axis=-1, keepdims=True)

        idx0 = idx_ref[:, 0:1]
        idx1 = idx_ref[:, 1:2]
        eids = lax.broadcasted_iota(jnp.int32, (N_TOK, N_EXP), 1)
        g0 = jnp.sum(jnp.where(eids == idx0, probs, 0.0), axis=1, keepdims=True)
        g1 = jnp.sum(jnp.where(eids == idx1, probs, 0.0), axis=1, keepdims=True)
        gs = g0 + g1
        g0 = g0 / gs
        g1 = g1 / gs
        w_cols = []
        for j in range(E_LOCAL):
            e = my * E_LOCAL + j
            w_cols.append(jnp.where(idx0 == e, g0, 0.0)
                          + jnp.where(idx1 == e, g1, 0.0))
        w_ref[:, :] = jnp.concatenate(w_cols, axis=1)

        ew_dma.wait()
        ewb_ref[...] = ewf_ref[...].astype(jnp.bfloat16)
        ewcat = ewb_ref[...].reshape(E_LOCAL * D_IN, D_OUT)

        def partial_chunk(c):
            rows = pl.ds(c * CHUNK, CHUNK)
            xc = xv_ref[rows, :]
            wc = w_ref[rows, :]
            xcat = jnp.concatenate(
                [(wc[:, j:j + 1] * xc).astype(jnp.bfloat16)
                 for j in range(E_LOCAL)],
                axis=1,
            )
            return jnp.dot(xcat, ewcat, preferred_element_type=jnp.float32)

        def rdma(src, dst, ssem, rsem, dev):
            return pltpu.make_async_remote_copy(
                src_ref=src, dst_ref=dst, send_sem=ssem, recv_sem=rsem,
                device_id=(dev,), device_id_type=pl.DeviceIdType.MESH,
            )

        rs = []
        for k, q in enumerate(peers):
            rs_send_buf[k] = partial_chunk(q).astype(jnp.bfloat16)
            if k == 0:
                pl.semaphore_wait(barrier_sem, N_PEER)
            for h in range(2):
                r = rdma(rs_send_buf.at[k, pl.ds(h * HALF, HALF), :],
                         rs_recv_buf.at[2 - k, pl.ds(h * HALF, HALF), :],
                         rs_send_sems.at[2 * k + h],
                         rs_recv_sems.at[2 * (2 - k) + h],
                         q)
                r.start()
                rs.append(r)

        mine = partial_chunk(my)

        ag = []
        for h in range(2):
            my_rows = pl.ds(my * CHUNK + h * HALF, HALF)
            for j in range(N_PEER):
                rdma(rs_send_buf.at[j, pl.ds(h * HALF, HALF), :],
                     rs_recv_buf.at[j, pl.ds(h * HALF, HALF), :],
                     rs_send_sems.at[2 * j + h],
                     rs_recv_sems.at[2 * j + h],
                     my).wait_recv()
            red = mine[h * HALF:(h + 1) * HALF, :]
            for j in range(N_PEER):
                red = red + rs_recv_buf[j, pl.ds(h * HALF, HALF), :].astype(
                    jnp.float32)
            out_ref[my_rows, :] = red.astype(jnp.bfloat16)
            for k, q in enumerate(peers):
                r = rdma(out_ref.at[my_rows, :],
                         out_ref.at[my_rows, :],
                         ag_send_sems.at[2 * k + h],
                         ag_recv_sems.at[2 * (2 - k) + h],
                         q)
                r.start()
                ag.append(r)
        for r in rs:
            r.wait_send()

        for j in range(N_PEER):
            u = _mod(my + j + 1)
            for h in range(2):
                rows = pl.ds(u * CHUNK + h * HALF, HALF)
                rdma(out_ref.at[rows, :],
                     out_ref.at[rows, :],
                     ag_send_sems.at[2 * j + h],
                     ag_recv_sems.at[2 * j + h],
                     my).wait_recv()
        for r in ag:
            r.wait_send()

    return pl.pallas_call(
        body,
        out_shape=jax.ShapeDtypeStruct((N_TOK, D_OUT), jnp.bfloat16),
        in_specs=[
            pl.BlockSpec(memory_space=pl.ANY),
            pl.BlockSpec(memory_space=pltpu.VMEM),
            pl.BlockSpec(memory_space=pltpu.VMEM),
            pl.BlockSpec(memory_space=pl.ANY),
        ],
        out_specs=pl.BlockSpec(memory_space=pltpu.VMEM),
        scratch_shapes=[
            pltpu.VMEM((N_TOK, D_IN), jnp.float32),
            pltpu.VMEM((E_LOCAL, D_IN, D_OUT), jnp.float32),
            pltpu.VMEM((E_LOCAL, D_IN, D_OUT), jnp.bfloat16),
            pltpu.VMEM((N_TOK, E_LOCAL), jnp.float32),
            pltpu.VMEM((N_PEER, CHUNK, D_OUT), jnp.bfloat16),
            pltpu.VMEM((N_PEER, CHUNK, D_OUT), jnp.bfloat16),
            pltpu.SemaphoreType.DMA((2,)),
            pltpu.SemaphoreType.DMA((2 * N_PEER,)),
            pltpu.SemaphoreType.DMA((2 * N_PEER,)),
            pltpu.SemaphoreType.DMA((2 * N_PEER,)),
            pltpu.SemaphoreType.DMA((2 * N_PEER,)),
        ],
        compiler_params=pltpu.CompilerParams(collective_id=0),
    )(x, router_W, route_idx, expert_W)


# device time: 18024 ns/iter; 1.0117x vs baseline; 1.0117x over previous
import jax
import jax.numpy as jnp
from jax import lax
from jax.experimental import pallas as pl
from jax.experimental.pallas import tpu as pltpu

N_DEV = 4
N_TOK = 512
D_IN = 256
D_OUT = 512
N_EXP = 16
E_LOCAL = N_EXP // N_DEV
CHUNK = N_TOK // N_DEV
HALF = CHUNK // 2
N_PEER = N_DEV - 1


def _mod(v):
    return lax.rem(v + 2 * N_DEV, N_DEV)


def kernel(x, router_W, route_idx, expert_W):
    def body(x_ref, rw_ref, idx_ref, ew_ref, out_ref,
             ewb_ref, w_ref, rs_send_buf, rs_recv_buf,
             rs_send_sems, rs_recv_sems, ag_send_sems, ag_recv_sems):
        my = lax.axis_index("i")
        peers = [_mod(my + k + 1) for k in range(N_PEER)]

        barrier_sem = pltpu.get_barrier_semaphore()
        for q in peers:
            pl.semaphore_signal(
                barrier_sem, inc=1,
                device_id=(q,), device_id_type=pl.DeviceIdType.MESH,
            )

        xf = x_ref[:, :]
        scores = jnp.dot(xf, rw_ref[:, :], preferred_element_type=jnp.float32)
        s_max = jnp.max(scores, axis=-1, keepdims=True)
        p = jnp.exp(scores - s_max)
        probs = p / jnp.sum(p, axis=-1, keepdims=True)

        idx0 = idx_ref[:, 0:1]
        idx1 = idx_ref[:, 1:2]
        eids = lax.broadcasted_iota(jnp.int32, (N_TOK, N_EXP), 1)
        g0 = jnp.sum(jnp.where(eids == idx0, probs, 0.0), axis=1, keepdims=True)
        g1 = jnp.sum(jnp.where(eids == idx1, probs, 0.0), axis=1, keepdims=True)
        gs = g0 + g1
        g0 = g0 / gs
        g1 = g1 / gs
        w_cols = []
        for j in range(E_LOCAL):
            e = my * E_LOCAL + j
            w_cols.append(jnp.where(idx0 == e, g0, 0.0)
                          + jnp.where(idx1 == e, g1, 0.0))
        w_ref[:, :] = jnp.concatenate(w_cols, axis=1)

        ewb_ref[...] = ew_ref[...].astype(jnp.bfloat16)
        ewcat = ewb_ref[...].reshape(E_LOCAL * D_IN, D_OUT)

        def partial_chunk(c):
            rows = pl.ds(c * CHUNK, CHUNK)
            xc = x_ref[rows, :]
            wc = w_ref[rows, :]
            xcat = jnp.concatenate(
                [(wc[:, j:j + 1] * xc).astype(jnp.bfloat16)
                 for j in range(E_LOCAL)],
                axis=1,
            )
            return jnp.dot(xcat, ewcat, preferred_element_type=jnp.float32)

        def rdma(src, dst, ssem, rsem, dev):
            return pltpu.make_async_remote_copy(
                src_ref=src, dst_ref=dst, send_sem=ssem, recv_sem=rsem,
                device_id=(dev,), device_id_type=pl.DeviceIdType.MESH,
            )

        rs = []
        for k, q in enumerate(peers):
            rs_send_buf[k] = partial_chunk(q).astype(jnp.bfloat16)
            if k == 0:
                pl.semaphore_wait(barrier_sem, N_PEER)
            for h in range(2):
                r = rdma(rs_send_buf.at[k, pl.ds(h * HALF, HALF), :],
                         rs_recv_buf.at[2 - k, pl.ds(h * HALF, HALF), :],
                         rs_send_sems.at[2 * k + h],
                         rs_recv_sems.at[2 * (2 - k) + h],
                         q)
                r.start()
                rs.append(r)

        mine = partial_chunk(my)

        ag = []
        for h in range(2):
            my_rows = pl.ds(my * CHUNK + h * HALF, HALF)
            for j in range(N_PEER):
                rdma(rs_send_buf.at[j, pl.ds(h * HALF, HALF), :],
                     rs_recv_buf.at[j, pl.ds(h * HALF, HALF), :],
                     rs_send_sems.at[2 * j + h],
                     rs_recv_sems.at[2 * j + h],
                     my).wait_recv()
            red = mine[h * HALF:(h + 1) * HALF, :]
            for j in range(N_PEER):
                red = red + rs_recv_buf[j, pl.ds(h * HALF, HALF), :].astype(
                    jnp.float32)
            out_ref[my_rows, :] = red.astype(jnp.bfloat16)
            for k, q in enumerate(peers):
                r = rdma(out_ref.at[my_rows, :],
                         out_ref.at[my_rows, :],
                         ag_send_sems.at[2 * k + h],
                         ag_recv_sems.at[2 * (2 - k) + h],
                         q)
                r.start()
                ag.append(r)
        for r in rs:
            r.wait_send()

        for j in range(N_PEER):
            u = _mod(my + j + 1)
            for h in range(2):
                rows = pl.ds(u * CHUNK + h * HALF, HALF)
                rdma(out_ref.at[rows, :],
                     out_ref.at[rows, :],
                     ag_send_sems.at[2 * j + h],
                     ag_recv_sems.at[2 * j + h],
                     my).wait_recv()
        for r in ag:
            r.wait_send()

    return pl.pallas_call(
        body,
        out_shape=jax.ShapeDtypeStruct((N_TOK, D_OUT), jnp.bfloat16),
        in_specs=[
            pl.BlockSpec(memory_space=pltpu.VMEM),
            pl.BlockSpec(memory_space=pltpu.VMEM),
            pl.BlockSpec(memory_space=pltpu.VMEM),
            pl.BlockSpec(memory_space=pltpu.VMEM),
        ],
        out_specs=pl.BlockSpec(memory_space=pltpu.VMEM),
        scratch_shapes=[
            pltpu.VMEM((E_LOCAL, D_IN, D_OUT), jnp.bfloat16),
            pltpu.VMEM((N_TOK, E_LOCAL), jnp.float32),
            pltpu.VMEM((N_PEER, CHUNK, D_OUT), jnp.bfloat16),
            pltpu.VMEM((N_PEER, CHUNK, D_OUT), jnp.bfloat16),
            pltpu.SemaphoreType.DMA((2 * N_PEER,)),
            pltpu.SemaphoreType.DMA((2 * N_PEER,)),
            pltpu.SemaphoreType.DMA((2 * N_PEER,)),
            pltpu.SemaphoreType.DMA((2 * N_PEER,)),
        ],
        compiler_params=pltpu.CompilerParams(collective_id=0),
    )(x, router_W, route_idx, expert_W)
